# baseline (speedup 1.0000x reference)
import functools

import jax
import jax.numpy as jnp
from jax import lax
from jax.experimental import pallas as pl
from jax.experimental.pallas import tpu as pltpu
from jax.experimental.pallas import tpu_sc as plsc

_LANES = 16


def _seg_sum_sc_body(nchunks, num_graphs, d_feat,
                     x_hbm, seg_hbm, out_hbm, yacc, rows_v, idx_v,
                     sem_seg, sem_x):
    rows_per_tile = num_graphs // 16

    cid = lax.axis_index("c")
    sid = lax.axis_index("s")
    wid = sid * 2 + cid

    zero = jnp.zeros((_LANES,), jnp.float32)

    def zbody(r, carry):
        for k in range(d_feat // _LANES):
            rows_v[0, r, pl.ds(k * _LANES, _LANES)] = zero
        return carry

    lax.fori_loop(0, rows_per_tile, zbody, 0)
    pltpu.sync_copy(rows_v.at[0, pl.ds(0, rows_per_tile)],
                    yacc.at[pl.ds(sid * rows_per_tile, rows_per_tile)])
    plsc.subcore_barrier()

    def start_gather(i):
        buf = i % 2
        c = wid * nchunks + i
        d1 = pltpu.async_copy(seg_hbm.at[c], idx_v.at[buf], sem_seg.at[buf])
        d2 = pltpu.async_copy(x_hbm.at[c], rows_v.at[buf], sem_x.at[buf])
        return (d1, d2)

    gd = [None] * nchunks
    gd[0] = start_gather(0)
    for i in range(nchunks):
        buf = i % 2
        if i + 1 < nchunks:
            gd[i + 1] = start_gather(i + 1)
        d1, d2 = gd[i]
        d1.wait()
        d2.wait()
        pltpu.sync_copy(rows_v.at[buf], yacc.at[idx_v.at[buf]], add=True)

    plsc.subcore_barrier()

    base = sid * rows_per_tile
    pltpu.sync_copy(yacc.at[pl.ds(base, rows_per_tile)],
                    rows_v.at[0, pl.ds(0, rows_per_tile)])
    pltpu.sync_copy(rows_v.at[0, pl.ds(0, rows_per_tile)],
                    out_hbm.at[cid, pl.ds(base, rows_per_tile)])


def _seg_sum_sc(x, seg32):
    n_nodes, d_feat = x.shape
    num_graphs = 512
    info = plsc.get_sparse_core_info()
    n_workers = info.num_cores * info.num_subcores
    rows_per_worker = n_nodes // n_workers
    assert rows_per_worker * n_workers == n_nodes
    chunk = max(d for d in range(1, 129) if rows_per_worker % d == 0)
    nchunks = rows_per_worker // chunk
    x3 = x.reshape(n_nodes // chunk, chunk, d_feat)
    seg2 = seg32.reshape(n_nodes // chunk, chunk)
    mesh = plsc.VectorSubcoreMesh(core_axis_name="c", subcore_axis_name="s")
    body = functools.partial(_seg_sum_sc_body, nchunks, num_graphs, d_feat)
    f = pl.kernel(
        body,
        out_type=jax.ShapeDtypeStruct((info.num_cores, num_graphs, d_feat),
                                      jnp.float32),
        mesh=mesh,
        scratch_types=[
            pltpu.VMEM_SHARED((num_graphs, d_feat), jnp.float32),
            pltpu.VMEM((2, chunk, d_feat), jnp.float32),
            pltpu.VMEM((2, chunk), jnp.int32),
            pltpu.SemaphoreType.DMA((2,)),
            pltpu.SemaphoreType.DMA((2,)),
        ],
    )
    return f(x3, seg2)


def _head_body(p_ref, w_ref, b_ref, o_ref):
    y = p_ref[0] + p_ref[1]
    y = jnp.maximum(y, 0.0)
    o_ref[...] = (
        jnp.dot(y, w_ref[...], preferred_element_type=jnp.float32)
        + b_ref[...])


def _head_tc(partials, W, b2):
    num_graphs = partials.shape[1]
    return pl.pallas_call(
        _head_body,
        out_shape=jax.ShapeDtypeStruct((num_graphs, W.shape[1]), jnp.float32),
    )(partials, W, b2)


def kernel(x, segment_ids, W, b):
    seg32 = segment_ids.astype(jnp.int32)
    partials = _seg_sum_sc(x, seg32)
    return _head_tc(partials, W, b.reshape(1, -1))


# speedup vs baseline: 1.0729x; 1.0729x over previous
import functools

import jax
import jax.numpy as jnp
from jax import lax
from jax.experimental import pallas as pl
from jax.experimental.pallas import tpu as pltpu
from jax.experimental.pallas import tpu_sc as plsc

_LANES = 16


def _seg_sum_sc_body(nchunks, num_graphs, d_feat, n_workers,
                     x_hbm, seg_hbm, out_hbm, yacc, rows_v, idx_v,
                     sem_seg, sem_x):
    rows_per_tile = num_graphs // 16

    cid = lax.axis_index("c")
    sid = lax.axis_index("s")
    wid = sid * 2 + cid

    zero = jnp.zeros((_LANES,), jnp.float32)

    def zbody(r, carry):
        for k in range(d_feat // _LANES):
            rows_v[0, r, pl.ds(k * _LANES, _LANES)] = zero
        return carry

    lax.fori_loop(0, rows_per_tile, zbody, 0)
    pltpu.sync_copy(rows_v.at[0, pl.ds(0, rows_per_tile)],
                    yacc.at[pl.ds(sid * rows_per_tile, rows_per_tile)])
    plsc.subcore_barrier()

    def start_gather(i):
        buf = i % 2
        c = wid + n_workers * i
        d1 = pltpu.async_copy(seg_hbm.at[c], idx_v.at[buf], sem_seg.at[buf])
        d2 = pltpu.async_copy(x_hbm.at[c], rows_v.at[buf], sem_x.at[buf])
        return (d1, d2)

    gd = [None] * nchunks
    gd[0] = start_gather(0)
    for i in range(nchunks):
        buf = i % 2
        if i + 1 < nchunks:
            gd[i + 1] = start_gather(i + 1)
        d1, d2 = gd[i]
        d1.wait()
        d2.wait()
        pltpu.sync_copy(rows_v.at[buf], yacc.at[idx_v.at[buf]], add=True)

    plsc.subcore_barrier()

    base = sid * rows_per_tile
    pltpu.sync_copy(yacc.at[pl.ds(base, rows_per_tile)],
                    rows_v.at[0, pl.ds(0, rows_per_tile)])
    pltpu.sync_copy(rows_v.at[0, pl.ds(0, rows_per_tile)],
                    out_hbm.at[cid, pl.ds(base, rows_per_tile)])


def _seg_sum_sc(x, seg32):
    n_nodes, d_feat = x.shape
    num_graphs = 512
    info = plsc.get_sparse_core_info()
    n_workers = info.num_cores * info.num_subcores
    rows_per_worker = n_nodes // n_workers
    assert rows_per_worker * n_workers == n_nodes
    chunk = max(d for d in range(1, 129) if rows_per_worker % d == 0)
    nchunks = rows_per_worker // chunk
    x3 = x.reshape(n_nodes // chunk, chunk, d_feat)
    seg2 = seg32.reshape(n_nodes // chunk, chunk)
    mesh = plsc.VectorSubcoreMesh(core_axis_name="c", subcore_axis_name="s")
    body = functools.partial(_seg_sum_sc_body, nchunks, num_graphs, d_feat,
                             n_workers)
    f = pl.kernel(
        body,
        out_type=jax.ShapeDtypeStruct((info.num_cores, num_graphs, d_feat),
                                      jnp.float32),
        mesh=mesh,
        scratch_types=[
            pltpu.VMEM_SHARED((num_graphs, d_feat), jnp.float32),
            pltpu.VMEM((2, chunk, d_feat), jnp.float32),
            pltpu.VMEM((2, chunk), jnp.int32),
            pltpu.SemaphoreType.DMA((2,)),
            pltpu.SemaphoreType.DMA((2,)),
        ],
    )
    return f(x3, seg2)


def _head_body(p_ref, w_ref, b_ref, o_ref):
    y = p_ref[0] + p_ref[1]
    y = jnp.maximum(y, 0.0)
    o_ref[...] = (
        jnp.dot(y, w_ref[...], preferred_element_type=jnp.float32)
        + b_ref[...])


def _head_tc(partials, W, b2):
    num_graphs = partials.shape[1]
    return pl.pallas_call(
        _head_body,
        out_shape=jax.ShapeDtypeStruct((num_graphs, W.shape[1]), jnp.float32),
    )(partials, W, b2)


def kernel(x, segment_ids, W, b):
    seg32 = segment_ids.astype(jnp.int32)
    partials = _seg_sum_sc(x, seg32)
    return _head_tc(partials, W, b.reshape(1, -1))
